# baseline (speedup 1.0000x reference)
import functools

import jax
import jax.numpy as jnp
from jax import lax
from jax.experimental import pallas as pl
from jax.experimental.pallas import tpu as pltpu
from jax.experimental.pallas import tpu_sc as plsc

N = 10000
E = 320000
D_IN = 128
D_HID = 16
N_CLS = 41

NC = 2
NS = 16
NW = NC * NS
EC = 128
K = 80
KA = 96
KB = 64
KMAX = max(KA, KB)
KD0 = 92
KD1 = 68
KDMAX = max(KD0, KD1)
NCHUNK = NW * K
NALLOC = NCHUNK + KMAX
E_PAD = NALLOC * EC
NP = 10112
RPT = NP // NS
D2 = 48

_mesh = plsc.VectorSubcoreMesh(core_axis_name="c", subcore_axis_name="s")


def _make_deg_kernel():
    @functools.partial(
        pl.kernel,
        mesh=_mesh,
        compiler_params=pltpu.CompilerParams(use_tc_tiling_on_sc=False),
        out_type=jax.ShapeDtypeStruct((NC, NP, 16), jnp.float32),
        scratch_types=[
            pltpu.VMEM((KDMAX, EC), jnp.int32),
            pltpu.VMEM((EC, 16), jnp.float32),
            pltpu.VMEM_SHARED((NP, 16), jnp.float32),
            pltpu.SemaphoreType.DMA,
        ],
    )
    def deg_kernel(dst_hbm, ones_hbm, zeros_hbm, out_hbm, dst_v, ones_v, acc, sem):
        c = lax.axis_index("c")
        s = lax.axis_index("s")
        r0 = s * RPT
        cbase = jnp.where(c == 0, s * KD0, NS * KD0 + s * KD1)
        pltpu.sync_copy(zeros_hbm.at[pl.ds(r0, RPT), :], acc.at[pl.ds(r0, RPT), :])
        pltpu.sync_copy(ones_hbm, ones_v)
        pltpu.sync_copy(dst_hbm.at[pl.ds(cbase, KDMAX), :], dst_v)
        plsc.subcore_barrier()

        GD = 4
        nd = jnp.where(c == 0, KD0 // GD, KD1 // GD)

        def body(j, carry):
            base = j * GD
            for b in range(GD):
                pltpu.async_copy(ones_v, acc.at[dst_v.at[base + b]], sem, add=True)

            @pl.when(j > 0)
            def _():
                for _b in range(GD):
                    pltpu.make_async_copy(ones_v, acc.at[dst_v.at[0]], sem).wait()

            return carry

        lax.fori_loop(0, nd, body, 0)
        for _b in range(GD):
            pltpu.make_async_copy(ones_v, acc.at[dst_v.at[0]], sem).wait()
        plsc.subcore_barrier()
        pltpu.sync_copy(acc.at[pl.ds(r0, RPT), :], out_hbm.at[c, pl.ds(r0, RPT), :])

    return deg_kernel


def _make_edge_kernel(d):
    SG = 4

    @functools.partial(
        pl.kernel,
        mesh=_mesh,
        compiler_params=pltpu.CompilerParams(use_tc_tiling_on_sc=False),
        out_type=jax.ShapeDtypeStruct((NC, NP, d), jnp.float32),
        scratch_types=[
            pltpu.VMEM((KMAX * EC,), jnp.int32),
            pltpu.VMEM((KMAX, EC), jnp.int32),
            [pltpu.VMEM((SG * EC, d), jnp.float32) for _ in range(2)],
            pltpu.VMEM_SHARED((NP, d), jnp.float32),
            pltpu.SemaphoreType.DMA,
            pltpu.SemaphoreType.DMA,
        ],
    )
    def edge_kernel(hs_hbm, src_hbm, dst_hbm, zeros_hbm, out_hbm,
                    src_v, dst_v, rows, acc, gsem, ssem):
        c = lax.axis_index("c")
        s = lax.axis_index("s")
        r0 = s * RPT
        cbase = jnp.where(c == 0, s * KA, NS * KA + s * KB)
        nsup2 = jnp.where(c == 0, KA // (2 * SG), KB // (2 * SG))
        nsup = 2 * nsup2
        @pl.when(c == 0)
        def _():
            pltpu.sync_copy(hs_hbm.at[pl.ds(r0, RPT), :], acc.at[pl.ds(r0, RPT), :])

        @pl.when(c != 0)
        def _():
            pltpu.sync_copy(zeros_hbm.at[pl.ds(r0, RPT), :], acc.at[pl.ds(r0, RPT), :])

        pltpu.sync_copy(src_hbm.at[pl.ds(cbase * EC, KMAX * EC)], src_v)
        pltpu.sync_copy(dst_hbm.at[pl.ds(cbase, KMAX), :], dst_v)
        plsc.subcore_barrier()

        SB = SG * EC
        pltpu.async_copy(hs_hbm.at[src_v.at[pl.ds(0, SB)]], rows[0], gsem)

        def body(gp, carry):
            for h in (0, 1):
                g = 2 * gp + h
                oh = 1 - h
                @pl.when(g >= 1)
                def _(oh=oh):
                    for b in range(SG):
                        pltpu.make_async_copy(
                            rows[oh].at[pl.ds(b * EC, EC), :],
                            acc.at[dst_v.at[0]], ssem).wait()
                pltpu.make_async_copy(
                    hs_hbm.at[src_v.at[pl.ds(0, SB)]], rows[h], gsem).wait()
                @pl.when(g + 1 < nsup)
                def _(g=g, oh=oh):
                    pltpu.async_copy(
                        hs_hbm.at[src_v.at[pl.ds((g + 1) * SB, SB)]],
                        rows[oh], gsem)
                for b in range(SG):
                    pltpu.async_copy(
                        rows[h].at[pl.ds(b * EC, EC), :],
                        acc.at[dst_v.at[g * SG + b]], ssem, add=True)
            return carry

        lax.fori_loop(0, nsup2, body, 0)
        for b in range(SG):
            pltpu.make_async_copy(
                rows[1].at[pl.ds(b * EC, EC), :], acc.at[dst_v.at[0]], ssem).wait()
        plsc.subcore_barrier()
        pltpu.sync_copy(acc.at[pl.ds(r0, RPT), :], out_hbm.at[c, pl.ds(r0, RPT), :])

    return edge_kernel


_deg_kernel = _make_deg_kernel()
_edge_kernel16 = _make_edge_kernel(D_HID)


def _tc_matmul1(feats_ref, w1_ref, h_ref):
    h_ref[...] = jnp.dot(feats_ref[...], w1_ref[...], preferred_element_type=jnp.float32)


def _tc_scale1(h_ref, degp_ref, hs1_ref, dinv_ref):
    deg = degp_ref[0] + degp_ref[1] + 1.0
    dinv = lax.rsqrt(deg)
    dinv_ref[...] = dinv
    hs1_ref[...] = h_ref[...] * dinv[:, 0:1]


def _tc_stage2(p_ref, dinv_ref, b1_ref, h1s_ref):
    dinv1 = dinv_ref[:, 0:1]
    h1 = jnp.maximum((p_ref[0] + p_ref[1]) * dinv1 + b1_ref[...], 0.0)
    h1s_ref[...] = h1 * dinv1


def _tc_stage3(p_ref, dinv_ref, w2_ref, b2_ref, out_ref):
    agg = (p_ref[0, 0:N, :] + p_ref[1, 0:N, :]) * dinv_ref[0:N, 0:1]
    out_ref[...] = (
        jnp.dot(agg, w2_ref[...], preferred_element_type=jnp.float32) + b2_ref[...]
    )


def kernel(feats, edge_index, W1, b1, W2, b2):
    f32 = jnp.float32
    feats_p = jnp.pad(feats, ((0, NP - N), (0, 0)))
    src = jnp.pad(edge_index[0], (0, E_PAD - E))
    dst = jnp.pad(edge_index[1], (0, E_PAD - E), constant_values=N)
    src_t = src
    dst_t = dst.reshape(NALLOC, EC)
    ones16 = jnp.ones((EC, 16), f32)
    zeros16 = jnp.zeros((NP, 16), f32)
    w2p = jnp.pad(W2, ((0, 0), (0, D2 - N_CLS)))
    b1r = b1.reshape(1, D_HID)
    b2r = jnp.pad(b2, (0, D2 - N_CLS)).reshape(1, D2)

    degp = _deg_kernel(dst_t, ones16, zeros16)
    h = pl.pallas_call(
        _tc_matmul1,
        out_shape=jax.ShapeDtypeStruct((NP, D_HID), f32),
    )(feats_p, W1)

    hs1, dinv = pl.pallas_call(
        _tc_scale1,
        out_shape=[
            jax.ShapeDtypeStruct((NP, D_HID), f32),
            jax.ShapeDtypeStruct((NP, 16), f32),
        ],
    )(h, degp)

    p1 = _edge_kernel16(hs1, src_t, dst_t, zeros16)

    h1s = pl.pallas_call(
        _tc_stage2,
        out_shape=jax.ShapeDtypeStruct((NP, D_HID), f32),
    )(p1, dinv, b1r)

    p2 = _edge_kernel16(h1s, src_t, dst_t, zeros16)

    out = pl.pallas_call(
        _tc_stage3,
        out_shape=jax.ShapeDtypeStruct((N, D2), f32),
    )(p2, dinv, w2p, b2r)

    return out[:, :N_CLS]


# speedup vs baseline: 1.0378x; 1.0378x over previous
import functools

import jax
import jax.numpy as jnp
from jax import lax
from jax.experimental import pallas as pl
from jax.experimental.pallas import tpu as pltpu
from jax.experimental.pallas import tpu_sc as plsc

N = 10000
E = 320000
D_IN = 128
D_HID = 16
N_CLS = 41

NC = 2
NS = 16
NW = NC * NS
EC = 128
K = 80
KA = 112
KB = 48
KMAX = max(KA, KB)
KD0 = 100
KD1 = 60
KDMAX = max(KD0, KD1)
NCHUNK = NW * K
NALLOC = NCHUNK + KMAX
E_PAD = NALLOC * EC
NP = 10112
RPT = NP // NS
D2 = 48

_mesh = plsc.VectorSubcoreMesh(core_axis_name="c", subcore_axis_name="s")


def _make_deg_kernel():
    @functools.partial(
        pl.kernel,
        mesh=_mesh,
        compiler_params=pltpu.CompilerParams(use_tc_tiling_on_sc=False),
        out_type=jax.ShapeDtypeStruct((NC, NP, 16), jnp.float32),
        scratch_types=[
            pltpu.VMEM((KDMAX, EC), jnp.int32),
            pltpu.VMEM((EC, 16), jnp.float32),
            pltpu.VMEM_SHARED((NP, 16), jnp.float32),
            pltpu.SemaphoreType.DMA,
        ],
    )
    def deg_kernel(dst_hbm, ones_hbm, zeros_hbm, out_hbm, dst_v, ones_v, acc, sem):
        c = lax.axis_index("c")
        s = lax.axis_index("s")
        r0 = s * RPT
        cbase = jnp.where(c == 0, s * KD0, NS * KD0 + s * KD1)
        pltpu.sync_copy(zeros_hbm.at[pl.ds(r0, RPT), :], acc.at[pl.ds(r0, RPT), :])
        pltpu.sync_copy(ones_hbm, ones_v)
        pltpu.sync_copy(dst_hbm.at[pl.ds(cbase, KDMAX), :], dst_v)
        plsc.subcore_barrier()

        GD = 4
        nd = jnp.where(c == 0, KD0 // GD, KD1 // GD)

        def body(j, carry):
            base = j * GD
            for b in range(GD):
                pltpu.async_copy(ones_v, acc.at[dst_v.at[base + b]], sem, add=True)

            @pl.when(j > 0)
            def _():
                for _b in range(GD):
                    pltpu.make_async_copy(ones_v, acc.at[dst_v.at[0]], sem).wait()

            return carry

        lax.fori_loop(0, nd, body, 0)
        for _b in range(GD):
            pltpu.make_async_copy(ones_v, acc.at[dst_v.at[0]], sem).wait()
        plsc.subcore_barrier()
        pltpu.sync_copy(acc.at[pl.ds(r0, RPT), :], out_hbm.at[c, pl.ds(r0, RPT), :])

    return deg_kernel


def _make_edge_kernel(d):
    SG = 4

    @functools.partial(
        pl.kernel,
        mesh=_mesh,
        compiler_params=pltpu.CompilerParams(use_tc_tiling_on_sc=False),
        out_type=jax.ShapeDtypeStruct((NC, NP, d), jnp.float32),
        scratch_types=[
            pltpu.VMEM((KMAX * EC,), jnp.int32),
            pltpu.VMEM((KMAX, EC), jnp.int32),
            [pltpu.VMEM((SG * EC, d), jnp.float32) for _ in range(2)],
            pltpu.VMEM_SHARED((NP, d), jnp.float32),
            pltpu.SemaphoreType.DMA,
            pltpu.SemaphoreType.DMA,
        ],
    )
    def edge_kernel(hs_hbm, src_hbm, dst_hbm, zeros_hbm, out_hbm,
                    src_v, dst_v, rows, acc, gsem, ssem):
        c = lax.axis_index("c")
        s = lax.axis_index("s")
        r0 = s * RPT
        cbase = jnp.where(c == 0, s * KA, NS * KA + s * KB)
        nsup2 = jnp.where(c == 0, KA // (2 * SG), KB // (2 * SG))
        nsup = 2 * nsup2
        @pl.when(c == 0)
        def _():
            pltpu.sync_copy(hs_hbm.at[pl.ds(r0, RPT), :], acc.at[pl.ds(r0, RPT), :])

        @pl.when(c != 0)
        def _():
            pltpu.sync_copy(zeros_hbm.at[pl.ds(r0, RPT), :], acc.at[pl.ds(r0, RPT), :])

        pltpu.sync_copy(src_hbm.at[pl.ds(cbase * EC, KMAX * EC)], src_v)
        pltpu.sync_copy(dst_hbm.at[pl.ds(cbase, KMAX), :], dst_v)
        plsc.subcore_barrier()

        SB = SG * EC
        pltpu.async_copy(hs_hbm.at[src_v.at[pl.ds(0, SB)]], rows[0], gsem)

        def body(gp, carry):
            for h in (0, 1):
                g = 2 * gp + h
                oh = 1 - h
                @pl.when(g >= 1)
                def _(oh=oh):
                    for b in range(SG):
                        pltpu.make_async_copy(
                            rows[oh].at[pl.ds(b * EC, EC), :],
                            acc.at[dst_v.at[0]], ssem).wait()
                pltpu.make_async_copy(
                    hs_hbm.at[src_v.at[pl.ds(0, SB)]], rows[h], gsem).wait()
                @pl.when(g + 1 < nsup)
                def _(g=g, oh=oh):
                    pltpu.async_copy(
                        hs_hbm.at[src_v.at[pl.ds((g + 1) * SB, SB)]],
                        rows[oh], gsem)
                for b in range(SG):
                    pltpu.async_copy(
                        rows[h].at[pl.ds(b * EC, EC), :],
                        acc.at[dst_v.at[g * SG + b]], ssem, add=True)
            return carry

        lax.fori_loop(0, nsup2, body, 0)
        for b in range(SG):
            pltpu.make_async_copy(
                rows[1].at[pl.ds(b * EC, EC), :], acc.at[dst_v.at[0]], ssem).wait()
        plsc.subcore_barrier()
        pltpu.sync_copy(acc.at[pl.ds(r0, RPT), :], out_hbm.at[c, pl.ds(r0, RPT), :])

    return edge_kernel


_deg_kernel = _make_deg_kernel()
_edge_kernel16 = _make_edge_kernel(D_HID)


def _tc_matmul1(feats_ref, w1_ref, h_ref):
    h_ref[...] = jnp.dot(feats_ref[...], w1_ref[...], preferred_element_type=jnp.float32)


def _tc_scale1(h_ref, degp_ref, hs1_ref, dinv_ref):
    deg = degp_ref[0] + degp_ref[1] + 1.0
    dinv = lax.rsqrt(deg)
    dinv_ref[...] = dinv
    hs1_ref[...] = h_ref[...] * dinv[:, 0:1]


def _tc_stage2(p_ref, dinv_ref, b1_ref, h1s_ref):
    dinv1 = dinv_ref[:, 0:1]
    h1 = jnp.maximum((p_ref[0] + p_ref[1]) * dinv1 + b1_ref[...], 0.0)
    h1s_ref[...] = h1 * dinv1


def _tc_stage3(p_ref, dinv_ref, w2_ref, b2_ref, out_ref):
    agg = (p_ref[0, 0:N, :] + p_ref[1, 0:N, :]) * dinv_ref[0:N, 0:1]
    out_ref[...] = (
        jnp.dot(agg, w2_ref[...], preferred_element_type=jnp.float32) + b2_ref[...]
    )


def kernel(feats, edge_index, W1, b1, W2, b2):
    f32 = jnp.float32
    feats_p = jnp.pad(feats, ((0, NP - N), (0, 0)))
    src = jnp.pad(edge_index[0], (0, E_PAD - E))
    dst = jnp.pad(edge_index[1], (0, E_PAD - E), constant_values=N)
    src_t = src
    dst_t = dst.reshape(NALLOC, EC)
    ones16 = jnp.ones((EC, 16), f32)
    zeros16 = jnp.zeros((NP, 16), f32)
    w2p = jnp.pad(W2, ((0, 0), (0, D2 - N_CLS)))
    b1r = b1.reshape(1, D_HID)
    b2r = jnp.pad(b2, (0, D2 - N_CLS)).reshape(1, D2)

    degp = _deg_kernel(dst_t, ones16, zeros16)
    h = pl.pallas_call(
        _tc_matmul1,
        out_shape=jax.ShapeDtypeStruct((NP, D_HID), f32),
    )(feats_p, W1)

    hs1, dinv = pl.pallas_call(
        _tc_scale1,
        out_shape=[
            jax.ShapeDtypeStruct((NP, D_HID), f32),
            jax.ShapeDtypeStruct((NP, 16), f32),
        ],
    )(h, degp)

    p1 = _edge_kernel16(hs1, src_t, dst_t, zeros16)

    h1s = pl.pallas_call(
        _tc_stage2,
        out_shape=jax.ShapeDtypeStruct((NP, D_HID), f32),
    )(p1, dinv, b1r)

    p2 = _edge_kernel16(h1s, src_t, dst_t, zeros16)

    out = pl.pallas_call(
        _tc_stage3,
        out_shape=jax.ShapeDtypeStruct((N, D2), f32),
    )(p2, dinv, w2p, b2r)

    return out[:, :N_CLS]


# speedup vs baseline: 1.0712x; 1.0322x over previous
import functools

import jax
import jax.numpy as jnp
from jax import lax
from jax.experimental import pallas as pl
from jax.experimental.pallas import tpu as pltpu
from jax.experimental.pallas import tpu_sc as plsc

N = 10000
E = 320000
D_IN = 128
D_HID = 16
N_CLS = 41

NC = 2
NS = 16
NW = NC * NS
EC = 128
K = 80
KA = 128
KB = 32
KMAX = max(KA, KB)
KD0 = 120
KD1 = 40
KDMAX = max(KD0, KD1)
NCHUNK = NW * K
NALLOC = NCHUNK + KMAX
E_PAD = NALLOC * EC
NP = 10112
RPT = NP // NS
D2 = 48

_mesh = plsc.VectorSubcoreMesh(core_axis_name="c", subcore_axis_name="s")


def _make_deg_kernel():
    @functools.partial(
        pl.kernel,
        mesh=_mesh,
        compiler_params=pltpu.CompilerParams(use_tc_tiling_on_sc=False),
        out_type=jax.ShapeDtypeStruct((NC, NP, 16), jnp.float32),
        scratch_types=[
            pltpu.VMEM((KDMAX, EC), jnp.int32),
            pltpu.VMEM((EC, 16), jnp.float32),
            pltpu.VMEM_SHARED((NP, 16), jnp.float32),
            pltpu.SemaphoreType.DMA,
        ],
    )
    def deg_kernel(dst_hbm, ones_hbm, zeros_hbm, out_hbm, dst_v, ones_v, acc, sem):
        c = lax.axis_index("c")
        s = lax.axis_index("s")
        r0 = s * RPT
        cbase = jnp.where(c == 0, s * KD0, NS * KD0 + s * KD1)
        pltpu.sync_copy(zeros_hbm.at[pl.ds(r0, RPT), :], acc.at[pl.ds(r0, RPT), :])
        pltpu.sync_copy(ones_hbm, ones_v)
        pltpu.sync_copy(dst_hbm.at[pl.ds(cbase, KDMAX), :], dst_v)
        plsc.subcore_barrier()

        GD = 4
        nd = jnp.where(c == 0, KD0 // GD, KD1 // GD)

        def body(j, carry):
            base = j * GD
            for b in range(GD):
                pltpu.async_copy(ones_v, acc.at[dst_v.at[base + b]], sem, add=True)

            @pl.when(j > 0)
            def _():
                for _b in range(GD):
                    pltpu.make_async_copy(ones_v, acc.at[dst_v.at[0]], sem).wait()

            return carry

        lax.fori_loop(0, nd, body, 0)
        for _b in range(GD):
            pltpu.make_async_copy(ones_v, acc.at[dst_v.at[0]], sem).wait()
        plsc.subcore_barrier()
        pltpu.sync_copy(acc.at[pl.ds(r0, RPT), :], out_hbm.at[c, pl.ds(r0, RPT), :])

    return deg_kernel


def _make_edge_kernel(d):
    SG = 4

    @functools.partial(
        pl.kernel,
        mesh=_mesh,
        compiler_params=pltpu.CompilerParams(use_tc_tiling_on_sc=False),
        out_type=jax.ShapeDtypeStruct((NC, NP, d), jnp.float32),
        scratch_types=[
            pltpu.VMEM((KMAX * EC,), jnp.int32),
            pltpu.VMEM((KMAX, EC), jnp.int32),
            [pltpu.VMEM((SG * EC, d), jnp.float32) for _ in range(2)],
            pltpu.VMEM_SHARED((NP, d), jnp.float32),
            pltpu.SemaphoreType.DMA,
            pltpu.SemaphoreType.DMA,
        ],
    )
    def edge_kernel(hs_hbm, src_hbm, dst_hbm, zeros_hbm, out_hbm,
                    src_v, dst_v, rows, acc, gsem, ssem):
        c = lax.axis_index("c")
        s = lax.axis_index("s")
        r0 = s * RPT
        cbase = jnp.where(c == 0, s * KA, NS * KA + s * KB)
        nsup2 = jnp.where(c == 0, KA // (2 * SG), KB // (2 * SG))
        nsup = 2 * nsup2
        @pl.when(c == 0)
        def _():
            pltpu.sync_copy(hs_hbm.at[pl.ds(r0, RPT), :], acc.at[pl.ds(r0, RPT), :])

        @pl.when(c != 0)
        def _():
            pltpu.sync_copy(zeros_hbm.at[pl.ds(r0, RPT), :], acc.at[pl.ds(r0, RPT), :])

        pltpu.sync_copy(src_hbm.at[pl.ds(cbase * EC, KMAX * EC)], src_v)
        pltpu.sync_copy(dst_hbm.at[pl.ds(cbase, KMAX), :], dst_v)
        plsc.subcore_barrier()

        SB = SG * EC
        pltpu.async_copy(hs_hbm.at[src_v.at[pl.ds(0, SB)]], rows[0], gsem)

        def body(gp, carry):
            for h in (0, 1):
                g = 2 * gp + h
                oh = 1 - h
                @pl.when(g >= 1)
                def _(oh=oh):
                    for b in range(SG):
                        pltpu.make_async_copy(
                            rows[oh].at[pl.ds(b * EC, EC), :],
                            acc.at[dst_v.at[0]], ssem).wait()
                pltpu.make_async_copy(
                    hs_hbm.at[src_v.at[pl.ds(0, SB)]], rows[h], gsem).wait()
                @pl.when(g + 1 < nsup)
                def _(g=g, oh=oh):
                    pltpu.async_copy(
                        hs_hbm.at[src_v.at[pl.ds((g + 1) * SB, SB)]],
                        rows[oh], gsem)
                for b in range(SG):
                    pltpu.async_copy(
                        rows[h].at[pl.ds(b * EC, EC), :],
                        acc.at[dst_v.at[g * SG + b]], ssem, add=True)
            return carry

        lax.fori_loop(0, nsup2, body, 0)
        for b in range(SG):
            pltpu.make_async_copy(
                rows[1].at[pl.ds(b * EC, EC), :], acc.at[dst_v.at[0]], ssem).wait()
        plsc.subcore_barrier()
        pltpu.sync_copy(acc.at[pl.ds(r0, RPT), :], out_hbm.at[c, pl.ds(r0, RPT), :])

    return edge_kernel


_deg_kernel = _make_deg_kernel()
_edge_kernel16 = _make_edge_kernel(D_HID)


def _tc_matmul1(feats_ref, w1_ref, h_ref):
    h_ref[...] = jnp.dot(feats_ref[...], w1_ref[...], preferred_element_type=jnp.float32)


def _tc_scale1(h_ref, degp_ref, hs1_ref, dinv_ref):
    deg = degp_ref[0] + degp_ref[1] + 1.0
    dinv = lax.rsqrt(deg)
    dinv_ref[...] = dinv
    hs1_ref[...] = h_ref[...] * dinv[:, 0:1]


def _tc_stage2(p_ref, dinv_ref, b1_ref, h1s_ref):
    dinv1 = dinv_ref[:, 0:1]
    h1 = jnp.maximum((p_ref[0] + p_ref[1]) * dinv1 + b1_ref[...], 0.0)
    h1s_ref[...] = h1 * dinv1


def _tc_stage3(p_ref, dinv_ref, w2_ref, b2_ref, out_ref):
    agg = (p_ref[0, 0:N, :] + p_ref[1, 0:N, :]) * dinv_ref[0:N, 0:1]
    out_ref[...] = (
        jnp.dot(agg, w2_ref[...], preferred_element_type=jnp.float32) + b2_ref[...]
    )


def kernel(feats, edge_index, W1, b1, W2, b2):
    f32 = jnp.float32
    feats_p = jnp.pad(feats, ((0, NP - N), (0, 0)))
    src = jnp.pad(edge_index[0], (0, E_PAD - E))
    dst = jnp.pad(edge_index[1], (0, E_PAD - E), constant_values=N)
    src_t = src
    dst_t = dst.reshape(NALLOC, EC)
    ones16 = jnp.ones((EC, 16), f32)
    zeros16 = jnp.zeros((NP, 16), f32)
    w2p = jnp.pad(W2, ((0, 0), (0, D2 - N_CLS)))
    b1r = b1.reshape(1, D_HID)
    b2r = jnp.pad(b2, (0, D2 - N_CLS)).reshape(1, D2)

    degp = _deg_kernel(dst_t, ones16, zeros16)
    h = pl.pallas_call(
        _tc_matmul1,
        out_shape=jax.ShapeDtypeStruct((NP, D_HID), f32),
    )(feats_p, W1)

    hs1, dinv = pl.pallas_call(
        _tc_scale1,
        out_shape=[
            jax.ShapeDtypeStruct((NP, D_HID), f32),
            jax.ShapeDtypeStruct((NP, 16), f32),
        ],
    )(h, degp)

    p1 = _edge_kernel16(hs1, src_t, dst_t, zeros16)

    h1s = pl.pallas_call(
        _tc_stage2,
        out_shape=jax.ShapeDtypeStruct((NP, D_HID), f32),
    )(p1, dinv, b1r)

    p2 = _edge_kernel16(h1s, src_t, dst_t, zeros16)

    out = pl.pallas_call(
        _tc_stage3,
        out_shape=jax.ShapeDtypeStruct((N, D2), f32),
    )(p2, dinv, w2p, b2r)

    return out[:, :N_CLS]
